# Initial kernel scaffold; baseline (speedup 1.0000x reference)
import functools

import jax
import jax.numpy as jnp
from jax import lax
from jax.experimental import pallas as pl
from jax.experimental.pallas import tpu as pltpu
from jax.experimental.pallas import tpu_sc as plsc

B = 8
V = 100000
S = 1024 * 1024

NC = 2
NS = 16
L = 16
NW = NC * NS

TECS_PER_ROW = NW // B
S_PER_TEC = S // TECS_PER_ROW
SAMPLE_ITERS = S_PER_TEC // L

LOG2M = 14
M = 1 << LOG2M

_C1 = jnp.int32(-372640083)
_C2 = jnp.int32(1935933847)
_GOLD = jnp.int32(-1640531527)


def _mix(x):
    x = x + _GOLD
    x = x ^ lax.shift_right_logical(x, 16)
    x = x * _C1
    x = x ^ lax.shift_right_logical(x, 15)
    x = x * _C2
    x = x ^ lax.shift_right_logical(x, 15)
    return x


def _sc_body(x_hbm, out_hbm, xrow, table, accbuf):
    cid = lax.axis_index("c")
    sid = lax.axis_index("s")
    wid = sid * NC + cid
    row = wid // TECS_PER_ROW
    lane = lax.iota(jnp.int32, L)

    pltpu.sync_copy(x_hbm.at[row], xrow)

    def sum_body(i, acc):
        return acc + xrow[pl.ds(i * L, L)]

    tot_vec = lax.fori_loop(0, V // L, sum_body, jnp.zeros((L,), jnp.float32))
    total = jnp.sum(tot_vec)

    def zero_body(k, carry):
        table[pl.ds(k * L, L)] = jnp.zeros((L,), jnp.int32)
        return carry

    lax.fori_loop(0, M // L, zero_body, 0)

    scale = jnp.float32(M) / total

    def cdf_body(i, carry):
        v = xrow[pl.ds(i * L, L)]
        c = plsc.cumsum(v) + carry
        b = jnp.minimum((c * scale).astype(jnp.int32), M - 1)
        ids = i * L + lane + 1
        plsc.store_scatter(table, [b], ids)
        return jnp.max(c)

    lax.fori_loop(0, V // L, cdf_body, jnp.float32(0.0))

    def fill_body(k, m):
        t = table[pl.ds(k * L, L)]
        cm = jnp.maximum(plsc.cummax(t), m)
        table[pl.ds(k * L, L)] = cm
        return jnp.max(cm)

    lax.fori_loop(0, M // L, fill_body, jnp.int32(0))

    base = wid * S_PER_TEC

    def sample_body(i, acc):
        sidv = base + i * L + lane
        h = _mix(sidv)
        g = lax.shift_right_logical(h, 32 - LOG2M)
        p = plsc.load_gather(table, [g])
        return acc + jnp.minimum(p, V - 1)

    acc = lax.fori_loop(0, SAMPLE_ITERS, sample_body,
                        jnp.zeros((L,), jnp.int32))

    accbuf[...] = acc
    pltpu.sync_copy(accbuf, out_hbm.at[wid])


@jax.jit
def _sc_sample(x):
    call = pl.kernel(
        _sc_body,
        out_type=jax.ShapeDtypeStruct((NW, L), jnp.int32),
        mesh=plsc.VectorSubcoreMesh(core_axis_name="c", subcore_axis_name="s"),
        scratch_types=[
            pltpu.VMEM((V,), jnp.float32),
            pltpu.VMEM((M,), jnp.int32),
            pltpu.VMEM((L,), jnp.int32),
        ],
    )
    return call(x)


def kernel(x):
    parts = _sc_sample(x)
    total = jnp.sum(parts.astype(jnp.float32))
    return (total / jnp.float32(B * S)).astype(jnp.float32)


# speedup vs baseline: 5987.0903x; 5987.0903x over previous
import functools

import jax
import jax.numpy as jnp
import numpy as np
from jax import lax
from jax.experimental import pallas as pl
from jax.experimental.pallas import tpu as pltpu
from jax.experimental.pallas import tpu_sc as plsc

B = 8
V = 100000
S = 1024 * 1024

NC = 2
NS = 16
L = 16
NW = NC * NS

TECS_PER_ROW = NW // B
S_PER_TEC = S // TECS_PER_ROW
SAMPLE_ITERS = S_PER_TEC // L

LOG2M = 14
M = 1 << LOG2M

_C1 = np.int32(-372640083)
_C2 = np.int32(1935933847)
_GOLD = np.int32(-1640531527)


def _mix(x):
    x = x + _GOLD
    x = x ^ lax.shift_right_logical(x, 16)
    x = x * _C1
    x = x ^ lax.shift_right_logical(x, 15)
    x = x * _C2
    x = x ^ lax.shift_right_logical(x, 15)
    return x


def _sc_body(x_hbm, out_hbm, xrow, table, accbuf):
    cid = lax.axis_index("c")
    sid = lax.axis_index("s")
    wid = sid * NC + cid
    row = wid // TECS_PER_ROW
    lane = lax.iota(jnp.int32, L)

    pltpu.sync_copy(x_hbm.at[row], xrow)

    def sum_body(i, acc):
        return acc + xrow[pl.ds(i * L, L)]

    tot_vec = lax.fori_loop(0, V // L, sum_body, jnp.zeros((L,), jnp.float32))
    total = jnp.sum(tot_vec)

    def zero_body(k, carry):
        table[pl.ds(k * L, L)] = jnp.zeros((L,), jnp.int32)
        return carry

    lax.fori_loop(0, M // L, zero_body, 0)

    scale = jnp.full((L,), float(M), jnp.float32) / (jnp.zeros((L,), jnp.float32) + total)

    def cdf_body(i, carry):
        v = xrow[pl.ds(i * L, L)]
        c = plsc.cumsum(v) + carry
        b = jnp.minimum((c * scale).astype(jnp.int32), M - 1)
        ids = i * L + lane + 1
        plsc.store_scatter(table, [b], ids)
        return jnp.max(c)

    lax.fori_loop(0, V // L, cdf_body, jnp.float32(0.0))

    def fill_body(k, m):
        t = table[pl.ds(k * L, L)]
        cm = jnp.maximum(plsc.cummax(t), m)
        table[pl.ds(k * L, L)] = cm
        return jnp.max(cm)

    lax.fori_loop(0, M // L, fill_body, jnp.int32(0))

    base = wid * S_PER_TEC

    def sample_body(i, acc):
        sidv = base + i * L + lane
        h = _mix(sidv)
        g = lax.shift_right_logical(h, 32 - LOG2M)
        p = plsc.load_gather(table, [g])
        return acc + jnp.minimum(p, V - 1)

    acc = lax.fori_loop(0, SAMPLE_ITERS, sample_body,
                        jnp.zeros((L,), jnp.int32))

    accbuf[...] = acc
    pltpu.sync_copy(accbuf, out_hbm.at[wid])


@jax.jit
def _sc_sample(x):
    call = pl.kernel(
        _sc_body,
        out_type=jax.ShapeDtypeStruct((NW, L), jnp.int32),
        mesh=plsc.VectorSubcoreMesh(core_axis_name="c", subcore_axis_name="s"),
        compiler_params=pltpu.CompilerParams(needs_layout_passes=False),
        scratch_types=[
            pltpu.VMEM((V,), jnp.float32),
            pltpu.VMEM((M,), jnp.int32),
            pltpu.VMEM((L,), jnp.int32),
        ],
    )
    return call(x)


def kernel(x):
    parts = _sc_sample(x)
    total = jnp.sum(parts.astype(jnp.float32))
    return (total / jnp.float32(B * S)).astype(jnp.float32)


# speedup vs baseline: 12275.3144x; 2.0503x over previous
import jax
import jax.numpy as jnp
import numpy as np
from jax import lax
from jax.experimental import pallas as pl
from jax.experimental.pallas import tpu as pltpu
from jax.experimental.pallas import tpu_sc as plsc

B = 8
V = 100000
S = 1024 * 1024

NC = 2
NS = 16
L = 16
NW = NC * NS

TECS_PER_ROW = NW // B
S_PER_TEC = S // TECS_PER_ROW

NSEG = 32
SEG = V // NSEG

LOG2M = 14
M = 1 << LOG2M
TSEG = M // L
TSTRIDE = TSEG + 1
TWORDS = L * TSTRIDE

UNITS = S_PER_TEC // (2 * L)
UNROLL_S = 4
UNROLL_P = 5

_C1 = np.int32(-372640083)
_C2 = np.int32(1935933847)
_GOLD = np.int32(-1640531527)


def _mix(x):
    x = x + _GOLD
    x = x ^ lax.shift_right_logical(x, 16)
    x = x * _C1
    x = x ^ lax.shift_right_logical(x, 15)
    x = x * _C2
    x = x ^ lax.shift_right_logical(x, 15)
    return x


def _taddr(b):
    return b + lax.shift_right_logical(b, 10)


def _sc_body(x_hbm, out_hbm, xrow, table, accbuf):
    cid = lax.axis_index("c")
    sid = lax.axis_index("s")
    wid = sid * NC + cid
    row = wid // TECS_PER_ROW
    lane = lax.iota(jnp.int32, L)
    zf = jnp.zeros((L,), jnp.float32)
    zi = jnp.zeros((L,), jnp.int32)

    segA = lane * SEG
    segB = segA + 16 * SEG

    pltpu.sync_copy(x_hbm.at[row], xrow)

    def sum_body(i, carry):
        a, b = carry
        for j in range(UNROLL_P):
            k = i * UNROLL_P + j
            a = a + plsc.load_gather(xrow, [segA + k])
            b = b + plsc.load_gather(xrow, [segB + k])
        return a, b

    sA, sB = lax.fori_loop(0, SEG // UNROLL_P, sum_body, (zf, zf))

    cA = plsc.cumsum(sA)
    cB = plsc.cumsum(sB)
    lastA = jnp.max(cA)
    offA = cA - sA
    offB = cB - sB + lastA
    total = jnp.max(cB) + lastA

    scale = jnp.full((L,), float(M), jnp.float32) / (zf + total)

    def zero_body(i, carry):
        for j in range(UNROLL_P):
            k = i * UNROLL_P + j
            table[pl.ds(k * L, L)] = zi
        return carry

    lax.fori_loop(0, TWORDS // (L * UNROLL_P), zero_body, 0)

    def cdf_body(i, carry):
        a, b = carry
        for j in range(UNROLL_P):
            k = i * UNROLL_P + j
            a = a + plsc.load_gather(xrow, [segA + k])
            b = b + plsc.load_gather(xrow, [segB + k])
            ba = jnp.minimum((a * scale).astype(jnp.int32), M - 1)
            bb = jnp.minimum((b * scale).astype(jnp.int32), M - 1)
            plsc.store_scatter(table, [_taddr(ba)], segA + k + 1)
            plsc.store_scatter(table, [_taddr(bb)], segB + k + 1)
        return a, b

    lax.fori_loop(0, SEG // UNROLL_P, cdf_body, (offA, offB))

    tbase = lane * TSTRIDE

    def fill_body(i, m):
        for j in range(UNROLL_S):
            k = i * UNROLL_S + j
            t = plsc.load_gather(table, [tbase + k])
            m = jnp.maximum(m, t)
            plsc.store_scatter(table, [tbase + k], m)
        return m

    m_end = lax.fori_loop(0, TSEG // UNROLL_S, fill_body, zi)

    cm = plsc.cummax(m_end)
    accbuf[...] = cm
    offs = plsc.load_gather(accbuf, [jnp.maximum(lane - 1, 0)])
    offs = jnp.where(lane == 0, 0, offs)
    vcap = zi + (V - 1)

    def fill2_body(i, carry):
        for j in range(UNROLL_S):
            k = i * UNROLL_S + j
            t = plsc.load_gather(table, [tbase + k])
            t = jnp.minimum(jnp.maximum(t, offs), vcap)
            plsc.store_scatter(table, [tbase + k], t)
        return carry

    lax.fori_loop(0, TSEG // UNROLL_S, fill2_body, 0)

    base = wid * UNITS * L
    mask14 = jnp.int32(M - 1)

    def sample_body(i, carry):
        accs = list(carry)
        for j in range(UNROLL_S):
            u = i * UNROLL_S + j
            h = _mix(base + u * L + lane)
            g1 = lax.shift_right_logical(h, 32 - LOG2M)
            g2 = lax.shift_right_logical(h, 4) & mask14
            p1 = plsc.load_gather(table, [_taddr(g1)])
            p2 = plsc.load_gather(table, [_taddr(g2)])
            accs[j] = accs[j] + p1 + p2
        return tuple(accs)

    accs = lax.fori_loop(0, UNITS // UNROLL_S, sample_body,
                         (zi, zi, zi, zi))

    accbuf[...] = accs[0] + accs[1] + accs[2] + accs[3]
    pltpu.sync_copy(accbuf, out_hbm.at[wid])


@jax.jit
def _sc_sample(x):
    call = pl.kernel(
        _sc_body,
        out_type=jax.ShapeDtypeStruct((NW, L), jnp.int32),
        mesh=plsc.VectorSubcoreMesh(core_axis_name="c", subcore_axis_name="s"),
        compiler_params=pltpu.CompilerParams(needs_layout_passes=False),
        scratch_types=[
            pltpu.VMEM((V,), jnp.float32),
            pltpu.VMEM((TWORDS,), jnp.int32),
            pltpu.VMEM((L,), jnp.int32),
        ],
    )
    return call(x)


def kernel(x):
    parts = _sc_sample(x)
    total = jnp.sum(parts.astype(jnp.float32))
    return (total / jnp.float32(B * S)).astype(jnp.float32)


# speedup vs baseline: 12910.1253x; 1.0517x over previous
import jax
import jax.numpy as jnp
import numpy as np
from jax import lax
from jax.experimental import pallas as pl
from jax.experimental.pallas import tpu as pltpu
from jax.experimental.pallas import tpu_sc as plsc

B = 8
V = 100000
S = 1024 * 1024

NC = 2
NS = 16
L = 16
NW = NC * NS

TECS_PER_ROW = NW // B
S_PER_TEC = S // TECS_PER_ROW

NSEG = 32
SEG = V // NSEG

LOG2M = 14
M = 1 << LOG2M
TSEG = M // L
TSTRIDE = TSEG + 1
TWORDS = L * TSTRIDE

UNITS = S_PER_TEC // (4 * L)
UNROLL_S = 4
UNROLL_P = 5

_C1 = np.int32(-372640083)
_C2 = np.int32(1935933847)
_GOLD = np.int32(-1640531527)


def _mix(x):
    x = x + _GOLD
    x = x ^ lax.shift_right_logical(x, 16)
    x = x * _C1
    x = x ^ lax.shift_right_logical(x, 15)
    x = x * _C2
    x = x ^ lax.shift_right_logical(x, 15)
    return x


def _taddr(b):
    return b + lax.shift_right_logical(b, 10)


def _sc_body(x_hbm, out_hbm, xrow, table, accbuf):
    cid = lax.axis_index("c")
    sid = lax.axis_index("s")
    wid = sid * NC + cid
    row = wid // TECS_PER_ROW
    lane = lax.iota(jnp.int32, L)
    zf = jnp.zeros((L,), jnp.float32)
    zi = jnp.zeros((L,), jnp.int32)

    segA = lane * SEG
    segB = segA + 16 * SEG

    pltpu.sync_copy(x_hbm.at[row], xrow)

    def sum_body(i, carry):
        a, b = carry
        for j in range(UNROLL_P):
            k = i * UNROLL_P + j
            a = a + plsc.load_gather(xrow, [segA + k])
            b = b + plsc.load_gather(xrow, [segB + k])
        return a, b

    sA, sB = lax.fori_loop(0, SEG // UNROLL_P, sum_body, (zf, zf))

    cA = plsc.cumsum(sA)
    cB = plsc.cumsum(sB)
    lastA = jnp.max(cA)
    offA = cA - sA
    offB = cB - sB + lastA
    total = jnp.max(cB) + lastA

    scale = jnp.full((L,), float(M), jnp.float32) / (zf + total)

    def zero_body(i, carry):
        for j in range(UNROLL_P):
            k = i * UNROLL_P + j
            table[pl.ds(k * L, L)] = zi
        return carry

    lax.fori_loop(0, TWORDS // (L * UNROLL_P), zero_body, 0)

    def cdf_body(i, carry):
        a, b = carry
        for j in range(UNROLL_P):
            k = i * UNROLL_P + j
            a = a + plsc.load_gather(xrow, [segA + k])
            b = b + plsc.load_gather(xrow, [segB + k])
            ba = jnp.minimum((a * scale).astype(jnp.int32), M - 1)
            bb = jnp.minimum((b * scale).astype(jnp.int32), M - 1)
            plsc.store_scatter(table, [_taddr(ba)], segA + k + 1)
            plsc.store_scatter(table, [_taddr(bb)], segB + k + 1)
        return a, b

    lax.fori_loop(0, SEG // UNROLL_P, cdf_body, (offA, offB))

    tbase = lane * TSTRIDE

    def fill_body(i, m):
        for j in range(UNROLL_S):
            k = i * UNROLL_S + j
            t = plsc.load_gather(table, [tbase + k])
            m = jnp.maximum(m, t)
            plsc.store_scatter(table, [tbase + k], m)
        return m

    m_end = lax.fori_loop(0, TSEG // UNROLL_S, fill_body, zi)

    cm = plsc.cummax(m_end)
    accbuf[...] = cm
    offs = plsc.load_gather(accbuf, [jnp.maximum(lane - 1, 0)])
    offs = jnp.where(lane == 0, 0, offs)
    vcap = zi + (V - 1)

    def fill2_body(i, carry):
        for j in range(UNROLL_S):
            k = i * UNROLL_S + j
            t = plsc.load_gather(table, [tbase + k])
            t = jnp.minimum(jnp.maximum(t, offs), vcap)
            plsc.store_scatter(table, [tbase + k], t)
        return carry

    lax.fori_loop(0, TSEG // UNROLL_S, fill2_body, 0)

    base = wid * UNITS * L
    mask14 = jnp.int32(M - 1)

    def sample_body(i, carry):
        accs = list(carry)
        for j in range(UNROLL_S):
            u = i * UNROLL_S + j
            h = _mix(base + u * L + lane)
            h2 = h * _C1
            g1 = lax.shift_right_logical(h, 32 - LOG2M)
            g2 = lax.shift_right_logical(h, 4) & mask14
            g3 = lax.shift_right_logical(h2, 32 - LOG2M)
            g4 = lax.shift_right_logical(h2, 4) & mask14
            p1 = plsc.load_gather(table, [_taddr(g1)])
            p2 = plsc.load_gather(table, [_taddr(g2)])
            p3 = plsc.load_gather(table, [_taddr(g3)])
            p4 = plsc.load_gather(table, [_taddr(g4)])
            accs[j] = accs[j] + ((p1 + p2) + (p3 + p4))
        return tuple(accs)

    accs = lax.fori_loop(0, UNITS // UNROLL_S, sample_body,
                         (zi, zi, zi, zi))

    accbuf[...] = accs[0] + accs[1] + accs[2] + accs[3]
    pltpu.sync_copy(accbuf, out_hbm.at[wid])


@jax.jit
def _sc_sample(x):
    call = pl.kernel(
        _sc_body,
        out_type=jax.ShapeDtypeStruct((NW, L), jnp.int32),
        mesh=plsc.VectorSubcoreMesh(core_axis_name="c", subcore_axis_name="s"),
        compiler_params=pltpu.CompilerParams(needs_layout_passes=False),
        scratch_types=[
            pltpu.VMEM((V,), jnp.float32),
            pltpu.VMEM((TWORDS,), jnp.int32),
            pltpu.VMEM((L,), jnp.int32),
        ],
    )
    return call(x)


def kernel(x):
    parts = _sc_sample(x)
    total = jnp.sum(parts.astype(jnp.float32))
    return (total / jnp.float32(B * S)).astype(jnp.float32)


# speedup vs baseline: 19479.6305x; 1.5089x over previous
import jax
import jax.numpy as jnp
import numpy as np
from jax import lax
from jax.experimental import pallas as pl
from jax.experimental.pallas import tpu as pltpu
from jax.experimental.pallas import tpu_sc as plsc

B = 8
V = 100000
S = 1024 * 1024

NC = 2
NS = 16
L = 16
NW = NC * NS

TECS_PER_ROW = NW // B
S_PER_TEC = S // TECS_PER_ROW

NSEG = 32
SEG = V // NSEG

LOG2M = 14
M = 1 << LOG2M
TSEG = M // L
TSTRIDE = TSEG + 1
TWORDS = L * TSTRIDE

UNITS = S_PER_TEC // (4 * L)

_C1 = np.int32(-372640083)
_C2 = np.int32(1935933847)
_GOLD = np.int32(-1640531527)


def _mix(x):
    x = x + _GOLD
    x = x ^ lax.shift_right_logical(x, 16)
    x = x * _C1
    x = x ^ lax.shift_right_logical(x, 15)
    x = x * _C2
    x = x ^ lax.shift_right_logical(x, 15)
    return x


def _taddr(b):
    return b + lax.shift_right_logical(b, 10)


def _sc_body(x_hbm, out_hbm, xrow, table, accbuf):
    cid = lax.axis_index("c")
    sid = lax.axis_index("s")
    wid = sid * NC + cid
    row = wid // TECS_PER_ROW
    lane = lax.iota(jnp.int32, L)
    zf = jnp.zeros((L,), jnp.float32)
    zi = jnp.zeros((L,), jnp.int32)

    segA = lane * SEG
    segB = segA + 16 * SEG

    pltpu.sync_copy(x_hbm.at[row], xrow)

    @plsc.parallel_loop(0, SEG, unroll=5, carry=(zf, zf))
    def p1(k, carry):
        a, b = carry
        a = a + plsc.load_gather(xrow, [segA + k])
        b = b + plsc.load_gather(xrow, [segB + k])
        return a, b

    sA, sB = p1

    cA = plsc.cumsum(sA)
    cB = plsc.cumsum(sB)
    lastA = jnp.max(cA)
    offA = cA - sA
    offB = cB - sB + lastA
    total = jnp.max(cB) + lastA

    scale = jnp.full((L,), float(M), jnp.float32) / (zf + total)

    @plsc.parallel_loop(0, TWORDS // L, unroll=5)
    def _z(k):
        table[pl.ds(k * L, L)] = zi

    @plsc.parallel_loop(0, SEG, unroll=5, carry=(offA, offB))
    def p2(k, carry):
        a, b = carry
        a = a + plsc.load_gather(xrow, [segA + k])
        b = b + plsc.load_gather(xrow, [segB + k])
        ba = jnp.minimum((a * scale).astype(jnp.int32), M - 1)
        bb = jnp.minimum((b * scale).astype(jnp.int32), M - 1)
        plsc.store_scatter(table, [_taddr(ba)], segA + k + 1)
        plsc.store_scatter(table, [_taddr(bb)], segB + k + 1)
        return a, b

    del p2

    tbase = lane * TSTRIDE

    @plsc.parallel_loop(0, TSEG, unroll=8, carry=zi)
    def fill1(k, m):
        t = plsc.load_gather(table, [tbase + k])
        m = jnp.maximum(m, t)
        plsc.store_scatter(table, [tbase + k], m)
        return m

    m_end = fill1

    cm = plsc.cummax(m_end)
    accbuf[...] = cm
    offs = plsc.load_gather(accbuf, [jnp.maximum(lane - 1, 0)])
    offs = jnp.where(lane == 0, 0, offs)
    vcap = zi + (V - 1)

    @plsc.parallel_loop(0, TSEG, unroll=8)
    def fill2(k):
        t = plsc.load_gather(table, [tbase + k])
        t = jnp.minimum(jnp.maximum(t, offs), vcap)
        plsc.store_scatter(table, [tbase + k], t)

    base = wid * UNITS * L
    mask14 = jnp.int32(M - 1)

    @plsc.parallel_loop(0, UNITS, unroll=4, carry=zi)
    def acc(u, a):
        h = _mix(base + u * L + lane)
        h2 = h * _C1
        g1 = lax.shift_right_logical(h, 32 - LOG2M)
        g2 = lax.shift_right_logical(h, 4) & mask14
        g3 = lax.shift_right_logical(h2, 32 - LOG2M)
        g4 = lax.shift_right_logical(h2, 4) & mask14
        p1_ = plsc.load_gather(table, [_taddr(g1)])
        p2_ = plsc.load_gather(table, [_taddr(g2)])
        p3_ = plsc.load_gather(table, [_taddr(g3)])
        p4_ = plsc.load_gather(table, [_taddr(g4)])
        return a + ((p1_ + p2_) + (p3_ + p4_))

    accbuf[...] = acc
    pltpu.sync_copy(accbuf, out_hbm.at[wid])


@jax.jit
def _sc_sample(x):
    call = pl.kernel(
        _sc_body,
        out_type=jax.ShapeDtypeStruct((NW, L), jnp.int32),
        mesh=plsc.VectorSubcoreMesh(core_axis_name="c", subcore_axis_name="s"),
        compiler_params=pltpu.CompilerParams(needs_layout_passes=False),
        scratch_types=[
            pltpu.VMEM((V,), jnp.float32),
            pltpu.VMEM((TWORDS,), jnp.int32),
            pltpu.VMEM((L,), jnp.int32),
        ],
    )
    return call(x)


def kernel(x):
    parts = _sc_sample(x)
    total = jnp.sum(parts.astype(jnp.float32))
    return (total / jnp.float32(B * S)).astype(jnp.float32)


# speedup vs baseline: 20533.5742x; 1.0541x over previous
import jax
import jax.numpy as jnp
import numpy as np
from jax import lax
from jax.experimental import pallas as pl
from jax.experimental.pallas import tpu as pltpu
from jax.experimental.pallas import tpu_sc as plsc

B = 8
V = 100000
S = 1024 * 1024

NC = 2
NS = 16
L = 16
NW = NC * NS

TECS_PER_ROW = NW // B
S_PER_TEC = S // TECS_PER_ROW

NSEG = 32
SEG = V // NSEG

TSTRIDE = 1025
M = L * TSTRIDE
UNITS = S_PER_TEC // (4 * L)

_C1 = np.int32(-372640083)
_C2 = np.int32(1935933847)
_GOLD = np.int32(-1640531527)


def _mix(x):
    x = x + _GOLD
    x = x ^ lax.shift_right_logical(x, 16)
    x = x * _C1
    x = x ^ lax.shift_right_logical(x, 15)
    x = x * _C2
    x = x ^ lax.shift_right_logical(x, 15)
    return x


def _sc_body(x_hbm, out_hbm, xrow, table, accbuf):
    cid = lax.axis_index("c")
    sid = lax.axis_index("s")
    wid = sid * NC + cid
    row = wid // TECS_PER_ROW
    lane = lax.iota(jnp.int32, L)
    zf = jnp.zeros((L,), jnp.float32)
    zi = jnp.zeros((L,), jnp.int32)

    segA = lane * SEG
    segB = segA + 16 * SEG

    pltpu.sync_copy(x_hbm.at[row], xrow)

    @plsc.parallel_loop(0, SEG, unroll=5, carry=(zf, zf))
    def p1(k, carry):
        a, b = carry
        a = a + plsc.load_gather(xrow, [segA + k])
        b = b + plsc.load_gather(xrow, [segB + k])
        return a, b

    sA, sB = p1

    cA = plsc.cumsum(sA)
    cB = plsc.cumsum(sB)
    lastA = jnp.max(cA)
    offA = cA - sA
    offB = cB - sB + lastA
    total = jnp.max(cB) + lastA

    scale = jnp.full((L,), float(M), jnp.float32) / (zf + total)

    @plsc.parallel_loop(0, M // L, unroll=5)
    def _z(k):
        table[pl.ds(k * L, L)] = zi

    @plsc.parallel_loop(0, SEG, unroll=5, carry=(offA, offB))
    def p2(k, carry):
        a, b = carry
        a = a + plsc.load_gather(xrow, [segA + k])
        b = b + plsc.load_gather(xrow, [segB + k])
        ba = jnp.minimum((a * scale).astype(jnp.int32), M - 1)
        bb = jnp.minimum((b * scale).astype(jnp.int32), M - 1)
        plsc.store_scatter(table, [ba], segA + k + 1)
        plsc.store_scatter(table, [bb], segB + k + 1)
        return a, b

    del p2

    tbase = lane * TSTRIDE

    @plsc.parallel_loop(0, TSTRIDE, unroll=5, carry=zi)
    def fill1(k, m):
        t = plsc.load_gather(table, [tbase + k])
        m = jnp.maximum(m, t)
        plsc.store_scatter(table, [tbase + k], m)
        return m

    m_end = fill1

    cm = plsc.cummax(m_end)
    accbuf[...] = cm
    offs = plsc.load_gather(accbuf, [jnp.maximum(lane - 1, 0)])
    offs = jnp.where(lane == 0, 0, offs)
    vcap = zi + (V - 1)

    @plsc.parallel_loop(0, TSTRIDE, unroll=5)
    def fill2(k):
        t = plsc.load_gather(table, [tbase + k])
        t = jnp.minimum(jnp.maximum(t, offs), vcap)
        plsc.store_scatter(table, [tbase + k], t)

    base = wid * UNITS * L
    mask16 = jnp.int32(0xFFFF)
    mconst = jnp.int32(M)

    @plsc.parallel_loop(0, UNITS, unroll=4, carry=zi)
    def acc(u, a):
        h = _mix(base + u * L + lane)
        h2 = h * _C1
        g1 = lax.shift_right_logical(lax.shift_right_logical(h, 16) * mconst, 16)
        g2 = lax.shift_right_logical((h & mask16) * mconst, 16)
        g3 = lax.shift_right_logical(lax.shift_right_logical(h2, 16) * mconst, 16)
        g4 = lax.shift_right_logical((h2 & mask16) * mconst, 16)
        p1_ = plsc.load_gather(table, [g1])
        p2_ = plsc.load_gather(table, [g2])
        p3_ = plsc.load_gather(table, [g3])
        p4_ = plsc.load_gather(table, [g4])
        return a + ((p1_ + p2_) + (p3_ + p4_))

    accbuf[...] = acc
    pltpu.sync_copy(accbuf, out_hbm.at[wid])


@jax.jit
def _sc_sample(x):
    call = pl.kernel(
        _sc_body,
        out_type=jax.ShapeDtypeStruct((NW, L), jnp.int32),
        mesh=plsc.VectorSubcoreMesh(core_axis_name="c", subcore_axis_name="s"),
        compiler_params=pltpu.CompilerParams(needs_layout_passes=False),
        scratch_types=[
            pltpu.VMEM((V,), jnp.float32),
            pltpu.VMEM((M,), jnp.int32),
            pltpu.VMEM((L,), jnp.int32),
        ],
    )
    return call(x)


def kernel(x):
    parts = _sc_sample(x)
    total = jnp.sum(parts.astype(jnp.float32))
    return (total / jnp.float32(B * S)).astype(jnp.float32)


# speedup vs baseline: 21390.7031x; 1.0417x over previous
import jax
import jax.numpy as jnp
import numpy as np
from jax import lax
from jax.experimental import pallas as pl
from jax.experimental.pallas import tpu as pltpu
from jax.experimental.pallas import tpu_sc as plsc

B = 8
V = 100000
S = 1024 * 1024

NC = 2
NS = 16
L = 16
NW = NC * NS

TECS_PER_ROW = NW // B
S_PER_TEC = S // TECS_PER_ROW

NSEG = 32
SEG = V // NSEG

TSTRIDE = 1025
M = L * TSTRIDE
UNITS = S_PER_TEC // (8 * L)

_C1 = np.int32(-372640083)
_C2 = np.int32(1935933847)
_GOLD = np.int32(-1640531527)


def _mix(x):
    x = x + _GOLD
    x = x ^ lax.shift_right_logical(x, 16)
    x = x * _C1
    x = x ^ lax.shift_right_logical(x, 15)
    x = x * _C2
    x = x ^ lax.shift_right_logical(x, 15)
    return x


def _sc_body(x_hbm, out_hbm, xrow, table, accbuf):
    cid = lax.axis_index("c")
    sid = lax.axis_index("s")
    wid = sid * NC + cid
    row = wid // TECS_PER_ROW
    lane = lax.iota(jnp.int32, L)
    zf = jnp.zeros((L,), jnp.float32)
    zi = jnp.zeros((L,), jnp.int32)

    segA = lane * SEG
    segB = segA + 16 * SEG

    pltpu.sync_copy(x_hbm.at[row], xrow)

    @plsc.parallel_loop(0, SEG, unroll=5, carry=(zf, zf))
    def p1(k, carry):
        a, b = carry
        a = a + plsc.load_gather(xrow, [segA + k])
        b = b + plsc.load_gather(xrow, [segB + k])
        return a, b

    sA, sB = p1

    cA = plsc.cumsum(sA)
    cB = plsc.cumsum(sB)
    lastA = jnp.max(cA)
    offA = cA - sA
    offB = cB - sB + lastA
    total = jnp.max(cB) + lastA

    scale = jnp.full((L,), float(M), jnp.float32) / (zf + total)

    @plsc.parallel_loop(0, M // L, unroll=5)
    def _z(k):
        table[pl.ds(k * L, L)] = zi

    @plsc.parallel_loop(0, SEG, unroll=5, carry=(offA, offB))
    def p2(k, carry):
        a, b = carry
        a = a + plsc.load_gather(xrow, [segA + k])
        b = b + plsc.load_gather(xrow, [segB + k])
        ba = jnp.minimum((a * scale).astype(jnp.int32), M - 1)
        bb = jnp.minimum((b * scale).astype(jnp.int32), M - 1)
        plsc.store_scatter(table, [ba], segA + k + 1)
        plsc.store_scatter(table, [bb], segB + k + 1)
        return a, b

    del p2

    tbase = lane * TSTRIDE

    @plsc.parallel_loop(0, TSTRIDE, unroll=5, carry=zi)
    def fill1(k, m):
        t = plsc.load_gather(table, [tbase + k])
        m = jnp.maximum(m, t)
        plsc.store_scatter(table, [tbase + k], m)
        return m

    m_end = fill1

    cm = plsc.cummax(m_end)
    accbuf[...] = cm
    offs = plsc.load_gather(accbuf, [jnp.maximum(lane - 1, 0)])
    offs = jnp.where(lane == 0, 0, offs)
    vcap = zi + (V - 1)

    @plsc.parallel_loop(0, TSTRIDE, unroll=5)
    def fill2(k):
        t = plsc.load_gather(table, [tbase + k])
        t = jnp.minimum(jnp.maximum(t, offs), vcap)
        plsc.store_scatter(table, [tbase + k], t)

    base = wid * UNITS * L
    mask16 = jnp.int32(0xFFFF)
    mconst = jnp.int32(M)

    @plsc.parallel_loop(0, UNITS, unroll=4, carry=zi)
    def acc(u, a):
        h = _mix(base + u * L + lane)
        hs = (h, h * _C1, h * _C2, (h * _C1) * _C2)
        ps = []
        for hw in hs:
            g_hi = lax.shift_right_logical(lax.shift_right_logical(hw, 16) * mconst, 16)
            g_lo = lax.shift_right_logical((hw & mask16) * mconst, 16)
            ps.append(plsc.load_gather(table, [g_hi]))
            ps.append(plsc.load_gather(table, [g_lo]))
        return a + (((ps[0] + ps[1]) + (ps[2] + ps[3]))
                    + ((ps[4] + ps[5]) + (ps[6] + ps[7])))

    accbuf[...] = acc
    pltpu.sync_copy(accbuf, out_hbm.at[wid])


@jax.jit
def _sc_sample(x):
    call = pl.kernel(
        _sc_body,
        out_type=jax.ShapeDtypeStruct((NW, L), jnp.int32),
        mesh=plsc.VectorSubcoreMesh(core_axis_name="c", subcore_axis_name="s"),
        compiler_params=pltpu.CompilerParams(needs_layout_passes=False),
        scratch_types=[
            pltpu.VMEM((V,), jnp.float32),
            pltpu.VMEM((M,), jnp.int32),
            pltpu.VMEM((L,), jnp.int32),
        ],
    )
    return call(x)


def kernel(x):
    parts = _sc_sample(x)
    total = jnp.sum(parts.astype(jnp.float32))
    return (total / jnp.float32(B * S)).astype(jnp.float32)


# speedup vs baseline: 23563.1327x; 1.1016x over previous
import jax
import jax.numpy as jnp
import numpy as np
from jax import lax
from jax.experimental import pallas as pl
from jax.experimental.pallas import tpu as pltpu
from jax.experimental.pallas import tpu_sc as plsc

B = 8
V = 100000
S = 1024 * 1024

NC = 2
NS = 16
L = 16
NW = NC * NS

TECS_PER_ROW = NW // B
S_PER_TEC = S // TECS_PER_ROW

NSEG = 32
SEG = V // NSEG

LOG2M = 14
M = 1 << LOG2M
UNITS = S_PER_TEC // (8 * L)

_C1 = np.int32(-372640083)
_C2 = np.int32(1935933847)
_GOLD = np.int32(-1640531527)


def _mix(x):
    x = x + _GOLD
    x = x ^ lax.shift_right_logical(x, 16)
    x = x * _C1
    x = x ^ lax.shift_right_logical(x, 15)
    x = x * _C2
    x = x ^ lax.shift_right_logical(x, 15)
    return x


def _sc_body(x_hbm, out_hbm, xrow, table, accbuf):
    cid = lax.axis_index("c")
    sid = lax.axis_index("s")
    wid = sid * NC + cid
    row = wid // TECS_PER_ROW
    lane = lax.iota(jnp.int32, L)
    zf = jnp.zeros((L,), jnp.float32)
    zi = jnp.zeros((L,), jnp.int32)

    segA = lane * SEG
    segB = segA + 16 * SEG

    pltpu.sync_copy(x_hbm.at[row], xrow)

    @plsc.parallel_loop(0, SEG, unroll=5, carry=(zf, zf))
    def p1(k, carry):
        a, b = carry
        a = a + plsc.load_gather(xrow, [segA + k])
        b = b + plsc.load_gather(xrow, [segB + k])
        return a, b

    sA, sB = p1

    cA = plsc.cumsum(sA)
    cB = plsc.cumsum(sB)
    lastA = jnp.max(cA)
    offA = cA - sA
    offB = cB - sB + lastA
    total = jnp.max(cB) + lastA

    scale = jnp.full((L,), float(M), jnp.float32) / (zf + total)

    @plsc.parallel_loop(0, M // L, unroll=4)
    def _z(k):
        table[pl.ds(k * L, L)] = zi

    @plsc.parallel_loop(0, SEG, unroll=5, carry=(offA, offB))
    def p2(k, carry):
        a, b = carry
        a = a + plsc.load_gather(xrow, [segA + k])
        b = b + plsc.load_gather(xrow, [segB + k])
        ba = jnp.minimum((a * scale).astype(jnp.int32), M - 1)
        bb = jnp.minimum((b * scale).astype(jnp.int32), M - 1)
        plsc.store_scatter(table, [ba], segA + k + 1)
        plsc.store_scatter(table, [bb], segB + k + 1)
        return a, b

    del p2

    vcap = zi + (V - 1)

    @plsc.parallel_loop(0, M // L, unroll=4, carry=zi)
    def fillx(r, run):
        v = table[pl.ds(r * L, L)]
        s = jnp.maximum(plsc.cummax(v), run)
        table[pl.ds(r * L, L)] = jnp.minimum(s, vcap)
        return jnp.maximum(run, zi + jnp.max(v))

    base = wid * UNITS * L
    mask14 = jnp.int32(M - 1)

    @plsc.parallel_loop(0, UNITS, unroll=4, carry=zi)
    def acc(u, a):
        h = _mix(base + u * L + lane)
        hs = (h, h * _C1, h * _C2, (h * _C1) * _C2)
        ps = []
        for hw in hs:
            g_hi = lax.shift_right_logical(hw, 32 - LOG2M)
            g_lo = lax.shift_right_logical(hw, 2) & mask14
            ps.append(plsc.load_gather(table, [g_hi]))
            ps.append(plsc.load_gather(table, [g_lo]))
        return a + (((ps[0] + ps[1]) + (ps[2] + ps[3]))
                    + ((ps[4] + ps[5]) + (ps[6] + ps[7])))

    accbuf[...] = acc
    pltpu.sync_copy(accbuf, out_hbm.at[wid])


@jax.jit
def _sc_sample(x):
    call = pl.kernel(
        _sc_body,
        out_type=jax.ShapeDtypeStruct((NW, L), jnp.int32),
        mesh=plsc.VectorSubcoreMesh(core_axis_name="c", subcore_axis_name="s"),
        compiler_params=pltpu.CompilerParams(needs_layout_passes=False),
        scratch_types=[
            pltpu.VMEM((V,), jnp.float32),
            pltpu.VMEM((M,), jnp.int32),
            pltpu.VMEM((L,), jnp.int32),
        ],
    )
    return call(x)


def kernel(x):
    parts = _sc_sample(x)
    total = jnp.sum(parts.astype(jnp.float32))
    return (total / jnp.float32(B * S)).astype(jnp.float32)


# speedup vs baseline: 26171.3006x; 1.1107x over previous
import jax
import jax.numpy as jnp
import numpy as np
from jax import lax
from jax.experimental import pallas as pl
from jax.experimental.pallas import tpu as pltpu
from jax.experimental.pallas import tpu_sc as plsc

B = 8
V = 100000
S = 1024 * 1024

NC = 2
NS = 16
L = 16
NW = NC * NS

Q = V // 4
SEGLO = Q // L

LOG2M = 14
M = 1 << LOG2M
UNITS = (B * S) // NW // (8 * L)

_C1 = np.int32(-372640083)
_C2 = np.int32(1935933847)
_GOLD = np.int32(-1640531527)


def _mix(x):
    x = x + _GOLD
    x = x ^ lax.shift_right_logical(x, 16)
    x = x * _C1
    x = x ^ lax.shift_right_logical(x, 15)
    x = x * _C2
    x = x ^ lax.shift_right_logical(x, 15)
    return x


def _sc_body(x_hbm, out_hbm, xq, table, tmpa, tmpb, tmpc, accbuf, qtbuf,
             sh_tot, sh_tab):
    cid = lax.axis_index("c")
    sid = lax.axis_index("s")
    w = cid * NS + sid
    grp = sid // 4
    q = sid % 4
    lane = lax.iota(jnp.int32, L)
    zf = jnp.zeros((L,), jnp.float32)
    zi = jnp.zeros((L,), jnp.int32)

    pltpu.sync_copy(x_hbm.at[(cid * 4 + grp) * 4 + q], xq)

    baseL = lane * SEGLO + jnp.minimum(lane, 8)
    tailmask = lane < 8

    @plsc.parallel_loop(0, M // L, unroll=4)
    def _z(k):
        table[pl.ds(k * L, L)] = zi

    @plsc.parallel_loop(0, SEGLO, unroll=11, carry=zf)
    def p1(k, a):
        return a + plsc.load_gather(xq, [baseL + k])

    tail = plsc.load_gather(xq, [jnp.minimum(baseL + SEGLO, Q - 1)])
    sL = p1 + jnp.where(tailmask, tail, 0.0)

    cL = plsc.cumsum(sL)
    offL = cL - sL
    qtot = jnp.max(cL)

    qtbuf[...] = zf + qtot
    pltpu.sync_copy(qtbuf, sh_tot.at[w])
    plsc.subcore_barrier()

    gbase = w - q
    tot = zf
    qoff = zf
    for t in range(4):
        pltpu.sync_copy(sh_tot.at[gbase + t], qtbuf)
        v = qtbuf[...]
        tot = tot + v
        qoff = qoff + jnp.where(jnp.int32(t) < q, v, 0.0)

    scale = jnp.full((L,), float(M), jnp.float32) / tot

    start = offL + qoff

    @plsc.parallel_loop(0, SEGLO, unroll=11, carry=start)
    def p2(k, a):
        a = a + plsc.load_gather(xq, [baseL + k])
        bb = jnp.minimum((a * scale).astype(jnp.int32), M - 1)
        plsc.store_scatter(table, [bb], baseL + k + 1)
        return a

    a_t = p2 + jnp.where(tailmask, tail, 0.0)
    b_t = jnp.minimum((a_t * scale).astype(jnp.int32), M - 1)
    plsc.store_scatter(table, [b_t], baseL + SEGLO + 1, mask=tailmask)

    @plsc.parallel_loop(0, M // L, unroll=4, carry=zi)
    def fillx(r, run):
        v = table[pl.ds(r * L, L)]
        table[pl.ds(r * L, L)] = jnp.maximum(plsc.cummax(v), run)
        return jnp.maximum(run, zi + jnp.max(v))

    pltpu.sync_copy(table, sh_tab.at[w])
    plsc.subcore_barrier()
    vcap = zi + (V - 1)
    for j, tmp in enumerate((tmpa, tmpb, tmpc)):
        src = gbase + lax.rem(q + (j + 1), 4)
        pltpu.sync_copy(sh_tab.at[src], tmp)

    @plsc.parallel_loop(0, M // L, unroll=4)
    def merge(r):
        s = pl.ds(r * L, L)
        tsum = ((table[s] + tmpa[s]) + (tmpb[s] + tmpc[s]))
        table[s] = jnp.minimum(tsum, vcap)

    base = w * UNITS * L
    mask14 = jnp.int32(M - 1)

    @plsc.parallel_loop(0, UNITS, unroll=4, carry=zi)
    def acc(u, a):
        h = _mix(base + u * L + lane)
        hs = (h, h * _C1, h * _C2, (h * _C1) * _C2)
        ps = []
        for hw in hs:
            g_hi = lax.shift_right_logical(hw, 32 - LOG2M)
            g_lo = lax.shift_right_logical(hw, 2) & mask14
            ps.append(plsc.load_gather(table, [g_hi]))
            ps.append(plsc.load_gather(table, [g_lo]))
        return a + (((ps[0] + ps[1]) + (ps[2] + ps[3]))
                    + ((ps[4] + ps[5]) + (ps[6] + ps[7])))

    accbuf[...] = acc
    pltpu.sync_copy(accbuf, out_hbm.at[w])


@jax.jit
def _sc_sample(x):
    call = pl.kernel(
        _sc_body,
        out_type=jax.ShapeDtypeStruct((NW, L), jnp.int32),
        mesh=plsc.VectorSubcoreMesh(core_axis_name="c", subcore_axis_name="s"),
        compiler_params=pltpu.CompilerParams(needs_layout_passes=False),
        scratch_types=[
            pltpu.VMEM((Q,), jnp.float32),
            pltpu.VMEM((M,), jnp.int32),
            pltpu.VMEM((M,), jnp.int32),
            pltpu.VMEM((M,), jnp.int32),
            pltpu.VMEM((M,), jnp.int32),
            pltpu.VMEM((L,), jnp.int32),
            pltpu.VMEM((L,), jnp.float32),
            pltpu.VMEM_SHARED((NW, L), jnp.float32),
            pltpu.VMEM_SHARED((NW, M), jnp.int32),
        ],
    )
    return call(x.reshape(NW, Q))


def kernel(x):
    parts = _sc_sample(x)
    total = jnp.sum(parts.astype(jnp.float32))
    return (total / jnp.float32(B * S)).astype(jnp.float32)
